# baseline (speedup 1.0000x reference)
import functools

import jax
import jax.numpy as jnp
from jax import lax
from jax.experimental import pallas as pl
from jax.experimental.pallas import tpu as pltpu
from jax.experimental.pallas import tpu_sc as plsc

_N = 100000
_E = 1600000
_GRAPHS = 64
_HID = 64

_N_PAD = 100352
_RB = 2048
_GRID = _N_PAD // _RB

_E_PAD = 1638400
_EROWS = 12800
_WROWS = _EROWS // 32
_SROWS = _EROWS // 16
_BLK = 16
_NBLK = _WROWS // _BLK
_NBLK_S = _SROWS // _BLK
_BLKL = 8
_NBLK_L = _SROWS // _BLKL

_RANGE = 25088
_ACC_ROWS = 25600
_ZROWS_PER_SUB = _ACC_ROWS // 16
_FLUSH_PER_SUB = _RANGE // 16
_FSUB_SMALL = _N_PAD // 16

def _worker_id():
    c = lax.axis_index("c")
    s = lax.axis_index("s")
    return c * 16 + s, c, s


def _deg_body(dst2d, ones4, zeros4, out, dstblk, ones_v, zer_v, acc):
    wid, c, s = _worker_id()
    pltpu.sync_copy(ones4, ones_v)
    pltpu.sync_copy(zeros4, zer_v)
    def zbody(i, _):
        pltpu.sync_copy(zer_v, acc.at[pl.ds(s * _FSUB_SMALL + i * 128, 128)])
        return 0
    lax.fori_loop(0, _FSUB_SMALL // 128, zbody, 0)
    plsc.subcore_barrier()

    def blk(b, _):
        row0 = wid * _WROWS + b * _BLK
        pltpu.sync_copy(dst2d.at[pl.ds(row0, _BLK)], dstblk)
        for g in range(_BLK):
            pltpu.sync_copy(ones_v, acc.at[dstblk.at[g]], add=True)
        return 0
    lax.fori_loop(0, _NBLK, blk, 0)
    plsc.subcore_barrier()
    pltpu.sync_copy(acc.at[pl.ds(s * _FSUB_SMALL, _FSUB_SMALL)],
                    out.at[c, pl.ds(s * _FSUB_SMALL, _FSUB_SMALL)])


_BLK4 = 8
_NPAIR4 = _WROWS // (2 * _BLK4)


def _agg4_body(src2d, dst2d, z_hbm, zeros4, out,
               srcAB, dstAB, rowsA, rowsB, zer_v,
               gsA, gsB, ssA, ssB, acc):
    wid, c, s = _worker_id()
    pltpu.sync_copy(zeros4, zer_v)
    def zbody(i, _):
        pltpu.sync_copy(zer_v, acc.at[pl.ds(s * _FSUB_SMALL + i * 128, 128)])
        return 0
    lax.fori_loop(0, _FSUB_SMALL // 128, zbody, 0)
    plsc.subcore_barrier()

    def pair(p, _):
        row0 = wid * _WROWS + p * 2 * _BLK4
        pltpu.sync_copy(src2d.at[pl.ds(row0, 2 * _BLK4)], srcAB)
        pltpu.sync_copy(dst2d.at[pl.ds(row0, 2 * _BLK4)], dstAB)
        gA = [pltpu.async_copy(z_hbm.at[srcAB.at[g]],
                               rowsA.at[pl.ds(g * 128, 128)], gsA)
              for g in range(_BLK4)]
        gB = [pltpu.async_copy(z_hbm.at[srcAB.at[_BLK4 + g]],
                               rowsB.at[pl.ds(g * 128, 128)], gsB)
              for g in range(_BLK4)]
        for d in gA:
            d.wait()
        dA = [pltpu.async_copy(rowsA.at[pl.ds(g * 128, 128)],
                               acc.at[dstAB.at[g]], ssA, add=True)
              for g in range(_BLK4)]
        for d in gB:
            d.wait()
        dB = [pltpu.async_copy(rowsB.at[pl.ds(g * 128, 128)],
                               acc.at[dstAB.at[_BLK4 + g]], ssB, add=True)
              for g in range(_BLK4)]
        for d in dA:
            d.wait()
        for d in dB:
            d.wait()
        return 0
    lax.fori_loop(0, _NPAIR4, pair, 0)
    plsc.subcore_barrier()
    pltpu.sync_copy(acc.at[pl.ds(s * _FSUB_SMALL, _FSUB_SMALL)],
                    out.at[c, pl.ds(s * _FSUB_SMALL, _FSUB_SMALL)])


_BLK64 = 4
_NPAIR64 = _SROWS // (2 * _BLK64)


def _agg64_body(src2d, dst2d, y_hbm, zeros64, out,
                srcblk, dstblk, siblk, rA, rB, rC,
                gs0, gs1, gs2, ss0, ss1, ss2, acc):
    wid, c, s = _worker_id()
    iota16 = lax.iota(jnp.int32, 16)
    bufs = [rA, rB, rC]
    gsems = [gs0, gs1, gs2]
    ssems = [ss0, ss1, ss2]

    for sweep in range(2):
        base = (2 * sweep + c) * _RANGE
        pltpu.sync_copy(zeros64, rA)
        def zbody(i, _):
            pltpu.sync_copy(rA, acc.at[pl.ds(s * _ZROWS_PER_SUB + i * 128, 128)])
            return 0
        lax.fori_loop(0, 12, zbody, 0)
        pltpu.sync_copy(rA.at[pl.ds(0, 64)],
                        acc.at[pl.ds(s * _ZROWS_PER_SUB + 12 * 128, 64)])
        plsc.subcore_barrier()

        def blk(b, _):
            row0 = s * _SROWS + b * _BLKL
            pltpu.sync_copy(src2d.at[pl.ds(row0, _BLKL)], srcblk)
            pltpu.sync_copy(dst2d.at[pl.ds(row0, _BLKL)], dstblk)
            gd = [None] * _BLKL
            sd = [None] * _BLKL
            gd[0] = pltpu.async_copy(y_hbm.at[srcblk.at[0]], bufs[0], gsems[0])
            gd[1] = pltpu.async_copy(y_hbm.at[srcblk.at[1]], bufs[1], gsems[1])
            for g in range(_BLKL):
                for sub in range(8):
                    d16 = dstblk[g, pl.ds(sub * 16, 16)]
                    loc = d16 - base
                    ok = (loc >= 0) & (loc < _RANGE)
                    spread = (d16 + iota16 + sub * 16) & 511
                    siblk[g, pl.ds(sub * 16, 16)] = jnp.where(
                        ok, loc, _RANGE + spread)
            for g in range(2, _BLKL):
                p = g % 3
                if g >= 3:
                    sd[g - 3].wait()
                gd[g] = pltpu.async_copy(y_hbm.at[srcblk.at[g]], bufs[p],
                                         gsems[p])
                q = (g - 2) % 3
                gd[g - 2].wait()
                sd[g - 2] = pltpu.async_copy(bufs[q], acc.at[siblk.at[g - 2]],
                                             ssems[q], add=True)
            for g in (_BLKL - 2, _BLKL - 1):
                q = g % 3
                gd[g].wait()
                sd[g] = pltpu.async_copy(bufs[q], acc.at[siblk.at[g]],
                                         ssems[q], add=True)
            sd[_BLKL - 3].wait()
            sd[_BLKL - 2].wait()
            sd[_BLKL - 1].wait()
            return 0
        lax.fori_loop(0, _NBLK_L, blk, 0)
        plsc.subcore_barrier()
        pltpu.sync_copy(
            acc.at[pl.ds(s * _FLUSH_PER_SUB, _FLUSH_PER_SUB)],
            out.at[pl.ds(base + s * _FLUSH_PER_SUB, _FLUSH_PER_SUB)])
        plsc.subcore_barrier()


@functools.lru_cache(maxsize=1)
def _sc_kernels():
    mesh = plsc.VectorSubcoreMesh(core_axis_name="c", subcore_axis_name="s")
    f32, i32 = jnp.float32, jnp.int32
    params = pltpu.CompilerParams(use_tc_tiling_on_sc=False,
                              vmem_limit_bytes=500000)
    deg = pl.kernel(
        _deg_body,
        compiler_params=params,
        out_type=jax.ShapeDtypeStruct((2, _N_PAD, 8), f32),
        mesh=mesh,
        scratch_types=[
            pltpu.VMEM((_BLK, 128), i32),
            pltpu.VMEM((128, 8), f32),
            pltpu.VMEM((128, 8), f32),
            pltpu.VMEM_SHARED((_N_PAD, 8), f32),
        ],
    )
    agg4 = pl.kernel(
        _agg4_body,
        compiler_params=params,
        out_type=jax.ShapeDtypeStruct((2, _N_PAD, 8), f32),
        mesh=mesh,
        scratch_types=[
            pltpu.VMEM((2 * _BLK4, 128), i32),
            pltpu.VMEM((2 * _BLK4, 128), i32),
            pltpu.VMEM((_BLK4 * 128, 8), f32),
            pltpu.VMEM((_BLK4 * 128, 8), f32),
            pltpu.VMEM((128, 8), f32),
            pltpu.SemaphoreType.DMA,
            pltpu.SemaphoreType.DMA,
            pltpu.SemaphoreType.DMA,
            pltpu.SemaphoreType.DMA,
            pltpu.VMEM_SHARED((_N_PAD, 8), f32),
        ],
    )
    agg64 = pl.kernel(
        _agg64_body,
        compiler_params=params,
        out_type=jax.ShapeDtypeStruct((_N_PAD, _HID), f32),
        mesh=mesh,
        scratch_types=[
            pltpu.VMEM((_BLKL, 128), i32),
            pltpu.VMEM((_BLKL, 128), i32),
            pltpu.VMEM((_BLKL, 128), i32),
            pltpu.VMEM((128, _HID), f32),
            pltpu.VMEM((128, _HID), f32),
            pltpu.VMEM((128, _HID), f32),
            pltpu.SemaphoreType.DMA,
            pltpu.SemaphoreType.DMA,
            pltpu.SemaphoreType.DMA,
            pltpu.SemaphoreType.DMA,
            pltpu.SemaphoreType.DMA,
            pltpu.SemaphoreType.DMA,
            pltpu.VMEM_SHARED((_ACC_ROWS, _HID), f32),
        ],
    )
    return deg, agg4, agg64


def _prep_body(degp, xp, dinv, z):
    deg = degp[0, :, 0:1] + degp[1, :, 0:1] + 1.0
    di = lax.rsqrt(deg)
    dinv[...] = di
    z[...] = xp[...] * di


def _h1y_body(s1p, z, dinv, w1, b1, y):
    di = dinv[...]
    s1 = (s1p[0] + s1p[1] + z[...]) * di
    h = jnp.maximum(jnp.dot(s1, w1[...], preferred_element_type=jnp.float32)
                    + b1[...], 0.0)
    y[...] = h * di


def _out_body(s2, y, dinv, batchp, w2, b2, wfc, bfc, out, sums, counts):
    i = pl.program_id(0)

    @pl.when(i == 0)
    def _():
        sums[...] = jnp.zeros_like(sums)
        counts[...] = jnp.zeros_like(counts)

    agg2 = (s2[...] + y[...]) * dinv[...]
    h2 = jnp.maximum(
        jnp.dot(agg2, w2[...], preferred_element_type=jnp.float32) + b2[...],
        0.0)
    onehot = (batchp[...] == lax.broadcasted_iota(jnp.int32, (_RB, _GRAPHS), 1)
              ).astype(jnp.float32)
    dn = (((0,), (0,)), ((), ()))
    sums[...] += lax.dot_general(onehot, h2, dn,
                                 preferred_element_type=jnp.float32)
    counts[...] += lax.dot_general(onehot, jnp.ones((_RB, 1), jnp.float32), dn,
                                   preferred_element_type=jnp.float32)

    @pl.when(i == _GRID - 1)
    def _():
        pooled = sums[...] / jnp.maximum(counts[...], 1.0)
        out[...] = (jnp.dot(pooled, wfc[...],
                            preferred_element_type=jnp.float32) + bfc[...])


def kernel(x, edge_index, batch, W1, b1, W2, b2, Wfc, bfc):
    f32 = jnp.float32
    src = edge_index[0]
    dst = edge_index[1]
    pad_e = _E_PAD - _E
    sent = jnp.full((pad_e,), _N, jnp.int32)
    src2d = jnp.concatenate([src, sent]).reshape(_EROWS, 128)
    dst2d = jnp.concatenate([dst, sent]).reshape(_EROWS, 128)

    xp = jnp.zeros((_N_PAD, 8), f32).at[:_N, :3].set(x)
    batchp = jnp.full((_N_PAD, 1), _GRAPHS, jnp.int32).at[:_N, 0].set(batch)
    w1p = jnp.zeros((8, _HID), f32).at[:3].set(W1)
    b1r = b1.reshape(1, _HID)
    b2r = b2.reshape(1, _HID)
    bfcr = bfc.reshape(1, -1)
    ones4 = jnp.ones((128, 8), f32)
    zeros4 = jnp.zeros((128, 8), f32)
    zeros64 = jnp.zeros((128, _HID), f32)

    _deg, _agg4, _agg64 = _sc_kernels()

    degp = _deg(dst2d, ones4, zeros4)

    dinv, z = pl.pallas_call(
        _prep_body,
        grid=(_GRID,),
        in_specs=[
            pl.BlockSpec((2, _RB, 8), lambda i: (0, i, 0)),
            pl.BlockSpec((_RB, 8), lambda i: (i, 0)),
        ],
        out_specs=[
            pl.BlockSpec((_RB, 1), lambda i: (i, 0)),
            pl.BlockSpec((_RB, 8), lambda i: (i, 0)),
        ],
        out_shape=[
            jax.ShapeDtypeStruct((_N_PAD, 1), f32),
            jax.ShapeDtypeStruct((_N_PAD, 8), f32),
        ],
    )(degp, xp)

    s1p = _agg4(src2d, dst2d, z, zeros4)

    y = pl.pallas_call(
        _h1y_body,
        grid=(_GRID,),
        in_specs=[
            pl.BlockSpec((2, _RB, 8), lambda i: (0, i, 0)),
            pl.BlockSpec((_RB, 8), lambda i: (i, 0)),
            pl.BlockSpec((_RB, 1), lambda i: (i, 0)),
            pl.BlockSpec((8, _HID), lambda i: (0, 0)),
            pl.BlockSpec((1, _HID), lambda i: (0, 0)),
        ],
        out_specs=pl.BlockSpec((_RB, _HID), lambda i: (i, 0)),
        out_shape=jax.ShapeDtypeStruct((_N_PAD, _HID), f32),
    )(s1p, z, dinv, w1p, b1r)

    s2 = _agg64(src2d, dst2d, y, zeros64)

    out = pl.pallas_call(
        _out_body,
        grid=(_GRID,),
        in_specs=[
            pl.BlockSpec((_RB, _HID), lambda i: (i, 0)),
            pl.BlockSpec((_RB, _HID), lambda i: (i, 0)),
            pl.BlockSpec((_RB, 1), lambda i: (i, 0)),
            pl.BlockSpec((_RB, 1), lambda i: (i, 0)),
            pl.BlockSpec((_HID, _HID), lambda i: (0, 0)),
            pl.BlockSpec((1, _HID), lambda i: (0, 0)),
            pl.BlockSpec((_HID, 5), lambda i: (0, 0)),
            pl.BlockSpec((1, 5), lambda i: (0, 0)),
        ],
        out_specs=pl.BlockSpec((_GRAPHS, 5), lambda i: (0, 0)),
        out_shape=jax.ShapeDtypeStruct((_GRAPHS, 5), f32),
        scratch_shapes=[
            pltpu.VMEM((_GRAPHS, _HID), f32),
            pltpu.VMEM((_GRAPHS, 1), f32),
        ],
    )(s2, y, dinv, batchp, W2, b2r, Wfc, bfcr)

    return out
